# Initial kernel scaffold; baseline (speedup 1.0000x reference)
import jax
import jax.numpy as jnp
from jax.experimental import pallas as pl


def kernel(bond_fea, angle_fea, nbr_idx, crys_idx, params):
    raise NotImplementedError("write your pallas kernel here")


# speedup vs baseline: 2.6255x; 2.6255x over previous
import functools

import jax
import jax.numpy as jnp
from jax.experimental import pallas as pl
from jax.experimental.pallas import tpu as pltpu
from jax.experimental.pallas import tpu_sc as plsc

_N = 2048
_M = 16
_S = 16
_NM = _N * _M
_NB = 64
_R = _NB * _M
_GRID = _N // _NB
_H = 128
_EPS = 1e-5


@functools.lru_cache(None)
def _sc_gather_fn(d, chunk):
  mesh = plsc.VectorSubcoreMesh(core_axis_name="c", subcore_axis_name="s")
  nw = 32
  bpw = _NM // nw

  def body(table, idx, out, idx_v, rows_v, sem):
    wid = jax.lax.axis_index("s") * 2 + jax.lax.axis_index("c")
    for c in range(bpw // chunk):
      base = wid * bpw + c * chunk
      pltpu.sync_copy(idx.at[pl.ds(base, chunk)], idx_v)
      pltpu.async_copy(table.at[idx_v], rows_v, sem).wait()
      pltpu.sync_copy(rows_v, out.at[pl.ds(base, chunk)])

  return pl.kernel(
      body, mesh=mesh,
      out_type=jax.ShapeDtypeStruct((_NM, d), jnp.float32),
      scratch_types=[pltpu.VMEM((chunk,), jnp.int32),
                     pltpu.VMEM((chunk, d), jnp.float32),
                     pltpu.SemaphoreType.DMA])


def _sc_gather(tab, idx):
  d = tab.shape[1]
  chunk = 1024 if d <= 64 else 256
  return _sc_gather_fn(d, chunk)(tab, idx)


def _softplus(x):
  m = jnp.maximum(x, 0.0)
  return m + jnp.log(jnp.exp(x - m) + jnp.exp(-m))


def _leaky(x):
  return jnp.where(x >= 0, x, 0.01 * x)


def _conv_pre(e_rows, eik, ang, prm):
  wl1t, wl23, wa1, wa23, blt, ba = prm[:6]
  x23 = jnp.concatenate([eik, ang], axis=1)
  lin = e_rows @ wl1t + x23 @ wl23 + blt
  att = e_rows @ wa1 + x23 @ wa23 + ba
  return att, lin


def _conv_edge(e_rows, eik, ang, prm, e16, gs, gk):
  g1t, b1t, g2, b2 = prm[6:]
  att, lin = _conv_pre(e_rows, eik, ang, prm)
  a = _leaky(att)
  a = a - jnp.max(a, axis=1, keepdims=True)
  ex = jnp.exp(a)
  alpha = ex / jnp.sum(ex, axis=1, keepdims=True)
  mu = (lin @ gs) * (1.0 / _S)
  xc = lin - mu @ e16
  var = ((xc * xc) @ gs) * (1.0 / _S)
  scale = alpha * jax.lax.rsqrt(alpha * alpha * var + _EPS)
  h = _softplus(xc * (scale @ e16) * g1t + b1t)
  out = e_rows + h @ gk
  mu2 = jnp.mean(out, axis=1, keepdims=True)
  var2 = jnp.mean((out - mu2) ** 2, axis=1, keepdims=True)
  return _softplus((out - mu2) * jax.lax.rsqrt(var2 + _EPS) * g2 + b2)


def _conv_ang(ang, e_rows, eik, prm, e16, gs):
  g2t, b2t = prm[6:]
  att, lin = _conv_pre(e_rows, eik, ang, prm)
  o = ang + (_leaky(att) @ e16) * lin
  mu = (o @ gs) * (1.0 / _S)
  xc = o - mu @ e16
  var = ((xc * xc) @ gs) * (1.0 / _S)
  return _softplus(xc * (jax.lax.rsqrt(var + _EPS) @ e16) * g2t + b2t)


def _gbf_fat(xfat, fvec, inv):
  return jnp.exp(-((xfat - fvec) ** 2) * inv)


def _pe0(bond2_ref, fe16_ref, out_ref):
  out_ref[...] = jnp.exp(-((bond2_ref[...] - fe16_ref[...]) ** 2) * 4.0)


def _pa(*refs):
  out_ref = refs[-1]
  vals = [r[...] for r in refs[:-1]]
  ang2, e0r, eik0 = vals[:3]
  prm = tuple(vals[3:13])
  e16, gs, gk, fa256 = vals[13:]
  ang0 = _gbf_fat(ang2 @ e16, fa256, 64.0)
  out_ref[...] = _conv_edge(e0r, eik0, ang0, prm, e16, gs, gk)


def _pb(*refs):
  out_ref = refs[-1]
  vals = [r[...] for r in refs[:-1]]
  ang2, e1r, eik1 = vals[:3]
  prma = tuple(vals[3:11])
  prme = tuple(vals[11:21])
  e16, gs, gk, fa256 = vals[21:]
  ang0 = _gbf_fat(ang2 @ e16, fa256, 64.0)
  ang1 = _conv_ang(ang0, e1r, eik1, prma, e16, gs)
  out_ref[...] = _conv_edge(e1r, eik1, ang1, prme, e16, gs, gk)


def _pc(*refs):
  out_ref = refs[-1]
  vals = [r[...] for r in refs[:-1]]
  ang2, e1r, e2r, eik1, eik2 = vals[:5]
  prma0 = tuple(vals[5:13])
  prma1 = tuple(vals[13:21])
  prme = tuple(vals[21:31])
  wex, bex, waxbd, baxt, gk2, bng, bnb, wo, bo = vals[31:40]
  e16, gs, gk, fa256 = vals[40:]
  ang0 = _gbf_fat(ang2 @ e16, fa256, 64.0)
  ang1 = _conv_ang(ang0, e1r, eik1, prma0, e16, gs)
  angf = _conv_ang(ang1, e2r, eik2, prma1, e16, gs)
  e3 = _conv_edge(e2r, eik2, angf, prme, e16, gs, gk)
  eh = _softplus(e3 @ wex + bex)
  ehs = jnp.sum(eh.reshape(_NB, _M, _H), axis=1)
  a4 = _softplus(angf @ waxbd + baxt)
  ak = _softplus(a4 @ gk2)
  ajs = jnp.sum(ak.reshape(_NB, _M, _H), axis=1)
  crys = jnp.concatenate([ehs, ajs], axis=1)
  mu = jnp.mean(crys, axis=1, keepdims=True)
  var = jnp.mean((crys - mu) ** 2, axis=1, keepdims=True)
  crys = _softplus((crys - mu) * jax.lax.rsqrt(var + _EPS) * bng + bnb)
  out_ref[...] = crys @ wo + bo


def _fs(x):
  nd = x.ndim
  return pl.BlockSpec(x.shape, lambda i, _n=nd: (0,) * _n)


def _conv_mats(p, kind):
  w = p['lin_w']
  wa = p['att_w']
  eye = jnp.eye(_S, dtype=jnp.float32)
  wl1t = jnp.tile(w[:_S], (1, _S))
  wl23 = jnp.concatenate([jnp.kron(eye, w[_S:2 * _S]),
                          jnp.kron(eye, w[2 * _S:])], axis=0)
  wa1 = wa[:_S]
  wa23 = jnp.concatenate([jnp.kron(eye, wa[_S:2 * _S]),
                          jnp.kron(eye, wa[2 * _S:])], axis=0)
  blt = jnp.tile(p['lin_b'].reshape(1, _S), (1, _S))
  ba = p['att_b'].reshape(1, 1)
  out = [wl1t, wl23, wa1, wa23, blt, ba]
  if kind == 'edge':
    out += [jnp.tile(p['bn1_g'].reshape(1, _S), (1, _S)),
            jnp.tile(p['bn1_b'].reshape(1, _S), (1, _S)),
            p['bn2_g'].reshape(1, _S), p['bn2_b'].reshape(1, _S)]
  else:
    out += [jnp.tile(p['bn2_g'].reshape(1, _S), (1, _S)),
            jnp.tile(p['bn2_b'].reshape(1, _S), (1, _S))]
  return out


def kernel(bond_fea, angle_fea, nbr_idx, crys_idx, params):
  del crys_idx
  bond2 = bond_fea.astype(jnp.float32).reshape(_NM, 1)
  ang2 = angle_fea.astype(jnp.float32).reshape(_NM, _M)
  nbrf = nbr_idx.astype(jnp.int32).reshape(_NM)

  eye = jnp.eye(_S, dtype=jnp.float32)
  e16 = jnp.kron(eye, jnp.ones((1, _S), jnp.float32))
  gs = jnp.kron(eye, jnp.ones((_S, 1), jnp.float32))
  gk = jnp.tile(eye, (_S, 1))
  gk2 = jnp.tile(jnp.eye(_H, dtype=jnp.float32), (_S, 1))
  fe16 = jnp.linspace(0.0, 8.0, _S, dtype=jnp.float32).reshape(1, _S)
  fe256 = jnp.tile(fe16, (1, _S))
  fa256 = jnp.tile(
      jnp.linspace(-1.0, 1.0, _S, dtype=jnp.float32).reshape(1, _S), (1, _S))
  consts2 = [e16, gs, gk, fa256]

  pe = [_conv_mats(p, 'edge') for p in params['edge_convs']]
  pa = [_conv_mats(p, 'ang') for p in params['ang_convs']]
  waxbd = jnp.kron(eye, params['expand_angle_w'])
  head = [params['expand_edge_w'], params['expand_edge_b'].reshape(1, _H),
          waxbd, jnp.tile(params['expand_angle_b'].reshape(1, _H), (1, _S)),
          gk2, params['bn_g'].reshape(1, 2 * _H), params['bn_b'].reshape(1, 2 * _H),
          params['out_w'], params['out_b'].reshape(1, 2)]

  cp = pltpu.CompilerParams(dimension_semantics=("parallel",))
  b_r1 = pl.BlockSpec((_R, 1), lambda i: (i, 0))
  b_r16 = pl.BlockSpec((_R, _S), lambda i: (i, 0))
  b_r256 = pl.BlockSpec((_R, _M * _S), lambda i: (i, 0))
  o_rows = jax.ShapeDtypeStruct((_NM, _S), jnp.float32)

  edge0 = pl.pallas_call(
      _pe0, grid=(_GRID,),
      in_specs=[b_r1, _fs(fe16)],
      out_specs=b_r16, out_shape=o_rows, compiler_params=cp)(bond2, fe16)

  eik0 = _sc_gather(edge0.reshape(_N, _M * _S), nbrf)

  ins = [ang2, edge0, eik0] + pe[0] + consts2
  edge1 = pl.pallas_call(
      _pa, grid=(_GRID,),
      in_specs=[b_r16, b_r16, b_r256] + [_fs(x) for x in pe[0] + consts2],
      out_specs=b_r16, out_shape=o_rows, compiler_params=cp)(*ins)

  eik1 = _sc_gather(edge1.reshape(_N, _M * _S), nbrf)

  ins = [ang2, edge1, eik1] + pa[0] + pe[1] + consts2
  edge2 = pl.pallas_call(
      _pb, grid=(_GRID,),
      in_specs=[b_r16, b_r16, b_r256] + [_fs(x) for x in pa[0] + pe[1] + consts2],
      out_specs=b_r16, out_shape=o_rows, compiler_params=cp)(*ins)

  eik2 = _sc_gather(edge2.reshape(_N, _M * _S), nbrf)

  ins = [ang2, edge1, edge2, eik1, eik2] + pa[0] + pa[1] + pe[2] + head + consts2
  out = pl.pallas_call(
      _pc, grid=(_GRID,),
      in_specs=[b_r16, b_r16, b_r16, b_r256, b_r256]
      + [_fs(x) for x in pa[0] + pa[1] + pe[2] + head + consts2],
      out_specs=pl.BlockSpec((_NB, 2), lambda i: (i, 0)),
      out_shape=jax.ShapeDtypeStruct((_N, 2), jnp.float32),
      compiler_params=cp)(*ins)
  return out
